# baseline (speedup 1.0000x reference)
import functools

import jax
import jax.numpy as jnp
from jax import lax
from jax.experimental import pallas as pl
from jax.experimental.pallas import tpu as pltpu
from jax.experimental.pallas import tpu_sc as plsc

N = 10000
E = 320000
D = 128
NS = 16
NC = 2
NPAD = 10240
WSEG = NPAD // NS
ECH = 128
NCHUNK = E // ECH
RPW = NPAD // NS
KB2 = 4

_MESH = plsc.VectorSubcoreMesh(
    core_axis_name="c", subcore_axis_name="s", num_cores=NC, num_subcores=NS)


def _deg_body(s0, d0, s1, d1, ones_hbm, zeros_hbm, deg_out,
              dout_sh, din_sh, ones_v, idx_m):
    c = lax.axis_index("c")
    s = lax.axis_index("s")
    seg = pl.ds(s * WSEG, WSEG)
    pltpu.sync_copy(zeros_hbm, dout_sh.at[seg])
    pltpu.sync_copy(zeros_hbm, din_sh.at[seg])
    pltpu.sync_copy(ones_hbm, ones_v)
    plsc.subcore_barrier()

    NGRP = NCHUNK // KB2

    def run(src_hbm, dst_hbm):
        ng = NGRP // NS + jnp.where(s < NGRP % NS, 1, 0)

        def body(i, carry):
            g = s + i * NS
            pltpu.sync_copy(src_hbm.at[pl.ds(g * KB2, KB2)], idx_m)
            for k in range(KB2):
                pltpu.sync_copy(ones_v, dout_sh.at[idx_m.at[k]], add=True)
            pltpu.sync_copy(dst_hbm.at[pl.ds(g * KB2, KB2)], idx_m)
            for k in range(KB2):
                pltpu.sync_copy(ones_v, din_sh.at[idx_m.at[k]], add=True)
            return carry

        lax.fori_loop(0, ng, body, 0)

    @pl.when(c == 0)
    def _():
        run(s0, d0)

    @pl.when(c == 1)
    def _():
        run(s1, d1)

    plsc.subcore_barrier()

    @pl.when(c == 0)
    def _():
        pltpu.sync_copy(dout_sh.at[seg], deg_out.at[0, seg])
        pltpu.sync_copy(din_sh.at[seg], deg_out.at[1, seg])

    @pl.when(c == 1)
    def _():
        pltpu.sync_copy(dout_sh.at[seg], deg_out.at[2, seg])
        pltpu.sync_copy(din_sh.at[seg], deg_out.at[3, seg])


@functools.partial(
    pl.kernel,
    out_type=jax.ShapeDtypeStruct((4, NPAD), jnp.float32),
    mesh=_MESH,
    scratch_types=[
        pltpu.VMEM_SHARED((NPAD,), jnp.float32),
        pltpu.VMEM_SHARED((NPAD,), jnp.float32),
        pltpu.VMEM((ECH,), jnp.float32),
        pltpu.VMEM((KB2, ECH), jnp.int32),
    ],
)
def _deg_call(*args):
    _deg_body(*args)


def _agg_body(h0, h1, s0, d0, s1, d1, z_hbm, agg_out,
              agg_sh, sidx, didx, rows, gs0, gs1, ss0, ss1):
    c = lax.axis_index("c")
    s = lax.axis_index("s")
    seg = pl.ds(s * RPW, RPW)
    pltpu.sync_copy(z_hbm, agg_sh.at[seg])
    plsc.subcore_barrier()

    def run(h_hbm, src_hbm, dst_hbm):
        nb = NCHUNK // NS + jnp.where(s < NCHUNK % NS, 1, 0)
        gsems = (gs0, gs1)
        ssems = (ss0, ss1)

        def load_idx(t, p):
            j = s + t * NS
            pltpu.sync_copy(src_hbm.at[j], sidx.at[p])
            pltpu.sync_copy(dst_hbm.at[j], didx.at[p])

        def start_gather(p):
            pltpu.async_copy(h_hbm.at[sidx.at[p]], rows.at[p], gsems[p])

        def wait_gather(p):
            pltpu.make_async_copy(h_hbm.at[pl.ds(0, ECH)], rows.at[p],
                                  gsems[p]).wait()

        def start_add(p):
            pltpu.async_copy(rows.at[p], agg_sh.at[didx.at[p]], ssems[p],
                             add=True)

        def wait_add(p):
            pltpu.make_async_copy(rows.at[p], agg_sh.at[pl.ds(0, ECH)],
                                  ssems[p]).wait()

        def add(p):
            pltpu.sync_copy(rows.at[p], agg_sh.at[didx.at[p]], add=True)

        load_idx(0, 0)
        start_gather(0)
        load_idx(1, 1)
        start_gather(1)

        def body(i, carry):
            t0 = 2 * i
            wait_gather(0)
            start_add(0)
            wait_gather(1)
            start_add(1)

            wait_add(0)

            @pl.when(t0 + 2 < nb)
            def _():
                load_idx(t0 + 2, 0)
                start_gather(0)

            wait_add(1)

            @pl.when(t0 + 3 < nb)
            def _():
                load_idx(t0 + 3, 1)
                start_gather(1)

            return carry

        lax.fori_loop(0, NCHUNK // NS // 2, body, 0)

        @pl.when(nb > NCHUNK // NS)
        def _():
            wait_gather(0)
            add(0)

    @pl.when(c == 0)
    def _():
        run(h0, s0, d0)

    @pl.when(c == 1)
    def _():
        run(h1, s1, d1)

    plsc.subcore_barrier()

    @pl.when(c == 0)
    def _():
        pltpu.sync_copy(agg_sh.at[seg], agg_out.at[0, seg])

    @pl.when(c == 1)
    def _():
        pltpu.sync_copy(agg_sh.at[seg], agg_out.at[1, seg])


@functools.partial(
    pl.kernel,
    out_type=jax.ShapeDtypeStruct((2, NPAD, D), jnp.float32),
    mesh=_MESH,
    scratch_types=[
        pltpu.VMEM_SHARED((NPAD, D), jnp.float32),
        pltpu.VMEM((2, ECH), jnp.int32),
        pltpu.VMEM((2, ECH), jnp.int32),
        pltpu.VMEM((2, ECH, D), jnp.float32),
        pltpu.SemaphoreType.DMA,
        pltpu.SemaphoreType.DMA,
        pltpu.SemaphoreType.DMA,
        pltpu.SemaphoreType.DMA,
    ],
)
def _agg_call(*args):
    _agg_body(*args)


_RB = 1000


def _h_body(x_ref, w0_ref, w1_ref, r0_ref, r1_ref, h0_ref, h1_ref):
    xb = x_ref[...]
    h0_ref[...] = jnp.dot(xb, w0_ref[...],
                          preferred_element_type=jnp.float32) * r0_ref[...]
    h1_ref[...] = jnp.dot(xb, w1_ref[...],
                          preferred_element_type=jnp.float32) * r1_ref[...]


def _h_call(x, w0, w1, r0c, r1c):
    grid = (N // _RB,)
    blk = pl.BlockSpec((_RB, D), lambda i: (i, 0))
    wblk = pl.BlockSpec((D, D), lambda i: (0, 0))
    cblk = pl.BlockSpec((_RB, 1), lambda i: (i, 0))
    return pl.pallas_call(
        _h_body,
        grid=grid,
        in_specs=[blk, wblk, wblk, cblk, cblk],
        out_specs=[blk, blk],
        out_shape=[jax.ShapeDtypeStruct((N, D), jnp.float32)] * 2,
    )(x, w0, w1, r0c, r1c)


def _fin_body(a0_ref, a1_ref, r0_ref, r1_ref, b0_ref, b1_ref, o_ref):
    o_ref[...] = 0.5 * ((a0_ref[...] * r0_ref[...] + b0_ref[...])
                        + (a1_ref[...] * r1_ref[...] + b1_ref[...]))


def _fin_call(a0, a1, r0c, r1c, b0, b1):
    grid = (N // _RB,)
    blk = pl.BlockSpec((_RB, D), lambda i: (i, 0))
    cblk = pl.BlockSpec((_RB, 1), lambda i: (i, 0))
    bblk = pl.BlockSpec((1, D), lambda i: (0, 0))
    return pl.pallas_call(
        _fin_body,
        grid=grid,
        in_specs=[blk, blk, cblk, cblk, bblk, bblk],
        out_specs=blk,
        out_shape=jax.ShapeDtypeStruct((N, D), jnp.float32),
    )(a0, a1, r0c, r1c, b0, b1)


def kernel(x, edge_index_0, edge_index_1, W0, b0, W1, b1):
    s0 = edge_index_0[0].reshape(NCHUNK, ECH)
    d0 = edge_index_0[1].reshape(NCHUNK, ECH)
    s1 = edge_index_1[0].reshape(NCHUNK, ECH)
    d1 = edge_index_1[1].reshape(NCHUNK, ECH)
    ones_hbm = jnp.ones((ECH,), jnp.float32)
    zeros_hbm = jnp.zeros((WSEG,), jnp.float32)
    z_hbm = jnp.zeros((RPW, D), jnp.float32)

    deg = _deg_call(s0, d0, s1, d1, ones_hbm, zeros_hbm)
    r = lax.rsqrt(jnp.maximum(deg[:, :N], 1.0))
    rout0 = r[0].reshape(N, 1)
    rin0 = r[1].reshape(N, 1)
    rout1 = r[2].reshape(N, 1)
    rin1 = r[3].reshape(N, 1)

    h0, h1 = _h_call(x, W0, W1, rout0, rout1)
    agg = _agg_call(h0, h1, s0, d0, s1, d1, z_hbm)
    return _fin_call(agg[0, :N], agg[1, :N], rin0, rin1,
                     b0.reshape(1, D), b1.reshape(1, D))


# speedup vs baseline: 1.1525x; 1.1525x over previous
import functools

import jax
import jax.numpy as jnp
from jax import lax
from jax.experimental import pallas as pl
from jax.experimental.pallas import tpu as pltpu
from jax.experimental.pallas import tpu_sc as plsc

N = 10000
E = 320000
D = 128
NS = 16
NC = 2
NPAD = 10240
WSEG = NPAD // NS
ECH = 128
NCHP = 2560
EPAD = NCHP * ECH
NB = NCHP // NS
RPW = NPAD // NS
KB = 8
NGRP = NCHP // KB
GSUB = NGRP // NS

_MESH = plsc.VectorSubcoreMesh(
    core_axis_name="c", subcore_axis_name="s", num_cores=NC, num_subcores=NS)


def _deg_body(s0, d0, s1, d1, ones_hbm, zeros_hbm, deg_out,
              dout_sh, din_sh, ones_v, six0, six1, dix0, dix1,
              ls0, ls1, as0, as1):
    c = lax.axis_index("c")
    s = lax.axis_index("s")
    seg = pl.ds(s * WSEG, WSEG)
    pltpu.sync_copy(zeros_hbm, dout_sh.at[seg])
    pltpu.sync_copy(zeros_hbm, din_sh.at[seg])
    pltpu.sync_copy(ones_hbm, ones_v)
    plsc.subcore_barrier()

    def run(src_hbm, dst_hbm):
        sixs = (six0, six1)
        dixs = (dix0, dix1)
        lsems = (ls0, ls1)
        asems = (as0, as1)

        def load(i, p):
            g = s + i * NS
            pltpu.async_copy(src_hbm.at[pl.ds(g * KB, KB)], sixs[p], lsems[p])
            pltpu.async_copy(dst_hbm.at[pl.ds(g * KB, KB)], dixs[p], lsems[p])

        def wait_load(p):
            for _ in range(2):
                pltpu.make_async_copy(src_hbm.at[pl.ds(0, KB)], sixs[p],
                                      lsems[p]).wait()

        def fire(p):
            for k in range(KB):
                pltpu.async_copy(ones_v, dout_sh.at[sixs[p].at[k]], asems[p],
                                 add=True)
                pltpu.async_copy(ones_v, din_sh.at[dixs[p].at[k]], asems[p],
                                 add=True)

        def drain(p):
            for _ in range(2 * KB):
                pltpu.make_async_copy(ones_v, dout_sh.at[pl.ds(0, ECH)],
                                      asems[p]).wait()

        load(0, 0)
        load(1, 1)

        def body(i, carry):
            t0 = 2 * i
            wait_load(0)
            fire(0)
            wait_load(1)
            fire(1)
            drain(0)

            @pl.when(t0 + 2 < GSUB)
            def _():
                load(t0 + 2, 0)

            drain(1)

            @pl.when(t0 + 3 < GSUB)
            def _():
                load(t0 + 3, 1)

            return carry

        lax.fori_loop(0, GSUB // 2, body, 0)

    @pl.when(c == 0)
    def _():
        run(s0, d0)

    @pl.when(c == 1)
    def _():
        run(s1, d1)

    plsc.subcore_barrier()

    @pl.when(c == 0)
    def _():
        pltpu.sync_copy(dout_sh.at[seg], deg_out.at[0, seg])
        pltpu.sync_copy(din_sh.at[seg], deg_out.at[1, seg])

    @pl.when(c == 1)
    def _():
        pltpu.sync_copy(dout_sh.at[seg], deg_out.at[2, seg])
        pltpu.sync_copy(din_sh.at[seg], deg_out.at[3, seg])


@functools.partial(
    pl.kernel,
    out_type=jax.ShapeDtypeStruct((4, NPAD), jnp.float32),
    mesh=_MESH,
    scratch_types=[
        pltpu.VMEM_SHARED((NPAD,), jnp.float32),
        pltpu.VMEM_SHARED((NPAD,), jnp.float32),
        pltpu.VMEM((ECH,), jnp.float32),
        pltpu.VMEM((KB, ECH), jnp.int32),
        pltpu.VMEM((KB, ECH), jnp.int32),
        pltpu.VMEM((KB, ECH), jnp.int32),
        pltpu.VMEM((KB, ECH), jnp.int32),
        pltpu.SemaphoreType.DMA,
        pltpu.SemaphoreType.DMA,
        pltpu.SemaphoreType.DMA,
        pltpu.SemaphoreType.DMA,
    ],
)
def _deg_call(*args):
    _deg_body(*args)


def _agg_body(h0, h1, s0, d0, s1, d1, z_hbm, agg_out,
              agg_sh, sidx, didx, rows, gs0, gs1):
    c = lax.axis_index("c")
    s = lax.axis_index("s")
    seg = pl.ds(s * RPW, RPW)
    pltpu.sync_copy(z_hbm, agg_sh.at[seg])
    plsc.subcore_barrier()

    def run(h_hbm, src_hbm, dst_hbm):
        gsems = (gs0, gs1)

        def load_idx(t, p):
            j = s + t * NS
            pltpu.sync_copy(src_hbm.at[j], sidx.at[p])
            pltpu.sync_copy(dst_hbm.at[j], didx.at[p])

        def start_gather(p):
            pltpu.async_copy(h_hbm.at[sidx.at[p]], rows.at[p], gsems[p])

        def wait_gather(p):
            pltpu.make_async_copy(h_hbm.at[pl.ds(0, ECH)], rows.at[p],
                                  gsems[p]).wait()

        def add(p):
            pltpu.sync_copy(rows.at[p], agg_sh.at[didx.at[p]], add=True)

        load_idx(0, 0)
        start_gather(0)
        load_idx(1, 1)
        start_gather(1)

        def body(i, carry):
            t0 = 2 * i
            wait_gather(0)
            add(0)

            @pl.when(t0 + 2 < NB)
            def _():
                load_idx(t0 + 2, 0)
                start_gather(0)

            wait_gather(1)
            add(1)

            @pl.when(t0 + 3 < NB)
            def _():
                load_idx(t0 + 3, 1)
                start_gather(1)

            return carry

        lax.fori_loop(0, NB // 2, body, 0)

    @pl.when(c == 0)
    def _():
        run(h0, s0, d0)

    @pl.when(c == 1)
    def _():
        run(h1, s1, d1)

    plsc.subcore_barrier()

    @pl.when(c == 0)
    def _():
        pltpu.sync_copy(agg_sh.at[seg], agg_out.at[0, seg])

    @pl.when(c == 1)
    def _():
        pltpu.sync_copy(agg_sh.at[seg], agg_out.at[1, seg])


@functools.partial(
    pl.kernel,
    out_type=jax.ShapeDtypeStruct((2, NPAD, D), jnp.float32),
    mesh=_MESH,
    scratch_types=[
        pltpu.VMEM_SHARED((NPAD, D), jnp.float32),
        pltpu.VMEM((2, ECH), jnp.int32),
        pltpu.VMEM((2, ECH), jnp.int32),
        pltpu.VMEM((2, ECH, D), jnp.float32),
        pltpu.SemaphoreType.DMA,
        pltpu.SemaphoreType.DMA,
    ],
)
def _agg_call(*args):
    _agg_body(*args)


_RB = 1000
_RBH = 1280


def _h_body(x_ref, w0_ref, w1_ref, r0_ref, r1_ref, h0_ref, h1_ref):
    xb = x_ref[...]
    h0_ref[...] = jnp.dot(xb, w0_ref[...],
                          preferred_element_type=jnp.float32) * r0_ref[...]
    h1_ref[...] = jnp.dot(xb, w1_ref[...],
                          preferred_element_type=jnp.float32) * r1_ref[...]


def _h_call(x, w0, w1, r0c, r1c):
    grid = (NPAD // _RBH,)
    blk = pl.BlockSpec((_RBH, D), lambda i: (i, 0))
    wblk = pl.BlockSpec((D, D), lambda i: (0, 0))
    cblk = pl.BlockSpec((_RBH, 1), lambda i: (i, 0))
    return pl.pallas_call(
        _h_body,
        grid=grid,
        in_specs=[blk, wblk, wblk, cblk, cblk],
        out_specs=[blk, blk],
        out_shape=[jax.ShapeDtypeStruct((NPAD, D), jnp.float32)] * 2,
    )(x, w0, w1, r0c, r1c)


def _fin_body(a0_ref, a1_ref, r0_ref, r1_ref, b0_ref, b1_ref, o_ref):
    o_ref[...] = 0.5 * ((a0_ref[...] * r0_ref[...] + b0_ref[...])
                        + (a1_ref[...] * r1_ref[...] + b1_ref[...]))


def _fin_call(a0, a1, r0c, r1c, b0, b1):
    grid = (N // _RB,)
    blk = pl.BlockSpec((_RB, D), lambda i: (i, 0))
    cblk = pl.BlockSpec((_RB, 1), lambda i: (i, 0))
    bblk = pl.BlockSpec((1, D), lambda i: (0, 0))
    return pl.pallas_call(
        _fin_body,
        grid=grid,
        in_specs=[blk, blk, cblk, cblk, bblk, bblk],
        out_specs=blk,
        out_shape=jax.ShapeDtypeStruct((N, D), jnp.float32),
    )(a0, a1, r0c, r1c, b0, b1)


def kernel(x, edge_index_0, edge_index_1, W0, b0, W1, b1):
    pad = (jnp.arange(EPAD - E, dtype=jnp.int32) % (NPAD - N)) + N
    s0 = jnp.concatenate([edge_index_0[0], pad]).reshape(NCHP, ECH)
    d0 = jnp.concatenate([edge_index_0[1], pad]).reshape(NCHP, ECH)
    s1 = jnp.concatenate([edge_index_1[0], pad]).reshape(NCHP, ECH)
    d1 = jnp.concatenate([edge_index_1[1], pad]).reshape(NCHP, ECH)
    xp = jnp.zeros((NPAD, D), jnp.float32).at[:N].set(x)
    ones_hbm = jnp.ones((ECH,), jnp.float32)
    zeros_hbm = jnp.zeros((WSEG,), jnp.float32)
    z_hbm = jnp.zeros((RPW, D), jnp.float32)

    deg = _deg_call(s0, d0, s1, d1, ones_hbm, zeros_hbm)
    r = lax.rsqrt(jnp.maximum(deg, 1.0))
    rout0 = r[0].reshape(NPAD, 1)
    rout1 = r[2].reshape(NPAD, 1)
    rin0 = r[1, :N].reshape(N, 1)
    rin1 = r[3, :N].reshape(N, 1)

    h0, h1 = _h_call(xp, W0, W1, rout0, rout1)
    agg = _agg_call(h0, h1, s0, d0, s1, d1, z_hbm)
    return _fin_call(agg[0], agg[1], rin0, rin1,
                     b0.reshape(1, D), b1.reshape(1, D))


# speedup vs baseline: 1.4404x; 1.2498x over previous
import functools

import jax
import jax.numpy as jnp
from jax import lax
from jax.experimental import pallas as pl
from jax.experimental.pallas import tpu as pltpu
from jax.experimental.pallas import tpu_sc as plsc

N = 10000
E = 320000
D = 128
NS = 16
NC = 2
NPAD = 10240
WSEG = NPAD // NS
ECH = 128
NCHP = 2560
EPAD = NCHP * ECH
NB = NCHP // NS
RPW = NPAD // NS
KB = 8
NGRP = NCHP // KB
GSUB = NGRP // NS

_MESH = plsc.VectorSubcoreMesh(
    core_axis_name="c", subcore_axis_name="s", num_cores=NC, num_subcores=NS)


def _deg_body(s0, d0, s1, d1, ones_hbm, zeros_hbm, deg_out,
              dout_sh, din_sh, ones_v, six0, six1, dix0, dix1,
              ls0, ls1, as0, as1):
    c = lax.axis_index("c")
    s = lax.axis_index("s")
    seg = pl.ds(s * WSEG, WSEG)
    pltpu.sync_copy(zeros_hbm, dout_sh.at[seg])
    pltpu.sync_copy(zeros_hbm, din_sh.at[seg])
    pltpu.sync_copy(ones_hbm, ones_v)
    plsc.subcore_barrier()

    def run(src_hbm, dst_hbm):
        sixs = (six0, six1)
        dixs = (dix0, dix1)
        lsems = (ls0, ls1)
        asems = (as0, as1)

        def load(i, p):
            g = s + i * NS
            pltpu.async_copy(src_hbm.at[pl.ds(g * KB, KB)], sixs[p], lsems[p])
            pltpu.async_copy(dst_hbm.at[pl.ds(g * KB, KB)], dixs[p], lsems[p])

        def wait_load(p):
            for _ in range(2):
                pltpu.make_async_copy(src_hbm.at[pl.ds(0, KB)], sixs[p],
                                      lsems[p]).wait()

        def fire(p):
            for k in range(KB):
                pltpu.async_copy(ones_v, dout_sh.at[sixs[p].at[k]], asems[p],
                                 add=True)
                pltpu.async_copy(ones_v, din_sh.at[dixs[p].at[k]], asems[p],
                                 add=True)

        def drain(p):
            for _ in range(2 * KB):
                pltpu.make_async_copy(ones_v, dout_sh.at[pl.ds(0, ECH)],
                                      asems[p]).wait()

        load(0, 0)
        load(1, 1)

        def body(i, carry):
            t0 = 2 * i
            wait_load(0)
            fire(0)
            wait_load(1)
            fire(1)
            drain(0)

            @pl.when(t0 + 2 < GSUB)
            def _():
                load(t0 + 2, 0)

            drain(1)

            @pl.when(t0 + 3 < GSUB)
            def _():
                load(t0 + 3, 1)

            return carry

        lax.fori_loop(0, GSUB // 2, body, 0)

    @pl.when(c == 0)
    def _():
        run(s0, d0)

    @pl.when(c == 1)
    def _():
        run(s1, d1)

    plsc.subcore_barrier()

    @pl.when(c == 0)
    def _():
        pltpu.sync_copy(dout_sh.at[seg], deg_out.at[0, seg])
        pltpu.sync_copy(din_sh.at[seg], deg_out.at[1, seg])

    @pl.when(c == 1)
    def _():
        pltpu.sync_copy(dout_sh.at[seg], deg_out.at[2, seg])
        pltpu.sync_copy(din_sh.at[seg], deg_out.at[3, seg])


@functools.partial(
    pl.kernel,
    out_type=jax.ShapeDtypeStruct((4, NPAD), jnp.float32),
    mesh=_MESH,
    scratch_types=[
        pltpu.VMEM_SHARED((NPAD,), jnp.float32),
        pltpu.VMEM_SHARED((NPAD,), jnp.float32),
        pltpu.VMEM((ECH,), jnp.float32),
        pltpu.VMEM((KB, ECH), jnp.int32),
        pltpu.VMEM((KB, ECH), jnp.int32),
        pltpu.VMEM((KB, ECH), jnp.int32),
        pltpu.VMEM((KB, ECH), jnp.int32),
        pltpu.SemaphoreType.DMA,
        pltpu.SemaphoreType.DMA,
        pltpu.SemaphoreType.DMA,
        pltpu.SemaphoreType.DMA,
    ],
)
def _deg_call(*args):
    _deg_body(*args)


def _agg_body(h0, h1, s0, d0, s1, d1, z_hbm, agg_out,
              agg_sh, sidx, didx, rows, gs0, gs1, lsa, lsb):
    c = lax.axis_index("c")
    s = lax.axis_index("s")
    seg = pl.ds(s * RPW, RPW)
    pltpu.sync_copy(z_hbm, agg_sh.at[seg])
    plsc.subcore_barrier()

    def run(h_hbm, src_hbm, dst_hbm):
        gsems = (gs0, gs1)

        def load_idx(t, q, lsem):
            j = s + t * NS
            pltpu.async_copy(src_hbm.at[j], sidx.at[q], lsem)
            pltpu.async_copy(dst_hbm.at[j], didx.at[q], lsem)

        def drain_idx(lsem):
            for _ in range(4):
                pltpu.make_async_copy(src_hbm.at[0], sidx.at[0], lsem).wait()

        def start_gather(q, p):
            pltpu.async_copy(h_hbm.at[sidx.at[q]], rows.at[p], gsems[p])

        def wait_gather(p):
            pltpu.make_async_copy(h_hbm.at[pl.ds(0, ECH)], rows.at[p],
                                  gsems[p]).wait()

        def add(p, q):
            pltpu.sync_copy(rows.at[p], agg_sh.at[didx.at[q]], add=True)

        for q in range(4):
            jq = s + q * NS
            pltpu.sync_copy(src_hbm.at[jq], sidx.at[q])
            pltpu.sync_copy(dst_hbm.at[jq], didx.at[q])
        start_gather(0, 0)
        start_gather(1, 1)

        def body(i, carry):
            c0 = 4 * i

            @pl.when(c0 > 0)
            def _():
                drain_idx(lsb)

            wait_gather(0)
            add(0, 0)
            start_gather(2, 0)
            wait_gather(1)
            add(1, 1)

            @pl.when(c0 + 4 < NB)
            def _():
                load_idx(c0 + 4, 0, lsa)
                load_idx(c0 + 5, 1, lsa)

            start_gather(3, 1)
            wait_gather(0)
            add(0, 2)

            @pl.when(c0 + 4 < NB)
            def _():
                drain_idx(lsa)
                start_gather(0, 0)

            wait_gather(1)
            add(1, 3)

            @pl.when(c0 + 6 < NB)
            def _():
                load_idx(c0 + 6, 2, lsb)
                load_idx(c0 + 7, 3, lsb)

            @pl.when(c0 + 5 < NB)
            def _():
                start_gather(1, 1)

            return carry

        lax.fori_loop(0, NB // 4, body, 0)

    @pl.when(c == 0)
    def _():
        run(h0, s0, d0)

    @pl.when(c == 1)
    def _():
        run(h1, s1, d1)

    plsc.subcore_barrier()

    @pl.when(c == 0)
    def _():
        pltpu.sync_copy(agg_sh.at[seg], agg_out.at[0, seg])

    @pl.when(c == 1)
    def _():
        pltpu.sync_copy(agg_sh.at[seg], agg_out.at[1, seg])


@functools.partial(
    pl.kernel,
    out_type=jax.ShapeDtypeStruct((2, NPAD, D), jnp.float32),
    mesh=_MESH,
    scratch_types=[
        pltpu.VMEM_SHARED((NPAD, D), jnp.float32),
        pltpu.VMEM((4, ECH), jnp.int32),
        pltpu.VMEM((4, ECH), jnp.int32),
        pltpu.VMEM((2, ECH, D), jnp.float32),
        pltpu.SemaphoreType.DMA,
        pltpu.SemaphoreType.DMA,
        pltpu.SemaphoreType.DMA,
        pltpu.SemaphoreType.DMA,
    ],
)
def _agg_call(*args):
    _agg_body(*args)


_RB = 1000
_RBH = 1280


def _h_body(x_ref, w0_ref, w1_ref, r0_ref, r1_ref, h0_ref, h1_ref):
    xb = x_ref[...]
    h0_ref[...] = jnp.dot(xb, w0_ref[...],
                          preferred_element_type=jnp.float32) * r0_ref[...]
    h1_ref[...] = jnp.dot(xb, w1_ref[...],
                          preferred_element_type=jnp.float32) * r1_ref[...]


def _h_call(x, w0, w1, r0c, r1c):
    grid = (NPAD // _RBH,)
    blk = pl.BlockSpec((_RBH, D), lambda i: (i, 0))
    wblk = pl.BlockSpec((D, D), lambda i: (0, 0))
    cblk = pl.BlockSpec((_RBH, 1), lambda i: (i, 0))
    return pl.pallas_call(
        _h_body,
        grid=grid,
        in_specs=[blk, wblk, wblk, cblk, cblk],
        out_specs=[blk, blk],
        out_shape=[jax.ShapeDtypeStruct((NPAD, D), jnp.float32)] * 2,
    )(x, w0, w1, r0c, r1c)


def _fin_body(a0_ref, a1_ref, r0_ref, r1_ref, b0_ref, b1_ref, o_ref):
    o_ref[...] = 0.5 * ((a0_ref[...] * r0_ref[...] + b0_ref[...])
                        + (a1_ref[...] * r1_ref[...] + b1_ref[...]))


def _fin_call(a0, a1, r0c, r1c, b0, b1):
    grid = (N // _RB,)
    blk = pl.BlockSpec((_RB, D), lambda i: (i, 0))
    cblk = pl.BlockSpec((_RB, 1), lambda i: (i, 0))
    bblk = pl.BlockSpec((1, D), lambda i: (0, 0))
    return pl.pallas_call(
        _fin_body,
        grid=grid,
        in_specs=[blk, blk, cblk, cblk, bblk, bblk],
        out_specs=blk,
        out_shape=jax.ShapeDtypeStruct((N, D), jnp.float32),
    )(a0, a1, r0c, r1c, b0, b1)


def kernel(x, edge_index_0, edge_index_1, W0, b0, W1, b1):
    pad = (jnp.arange(EPAD - E, dtype=jnp.int32) % (NPAD - N)) + N
    s0 = jnp.concatenate([edge_index_0[0], pad]).reshape(NCHP, ECH)
    d0 = jnp.concatenate([edge_index_0[1], pad]).reshape(NCHP, ECH)
    s1 = jnp.concatenate([edge_index_1[0], pad]).reshape(NCHP, ECH)
    d1 = jnp.concatenate([edge_index_1[1], pad]).reshape(NCHP, ECH)
    xp = jnp.zeros((NPAD, D), jnp.float32).at[:N].set(x)
    ones_hbm = jnp.ones((ECH,), jnp.float32)
    zeros_hbm = jnp.zeros((WSEG,), jnp.float32)
    z_hbm = jnp.zeros((RPW, D), jnp.float32)

    deg = _deg_call(s0, d0, s1, d1, ones_hbm, zeros_hbm)
    r = lax.rsqrt(jnp.maximum(deg, 1.0))
    rout0 = r[0].reshape(NPAD, 1)
    rout1 = r[2].reshape(NPAD, 1)
    rin0 = r[1, :N].reshape(N, 1)
    rin1 = r[3, :N].reshape(N, 1)

    h0, h1 = _h_call(xp, W0, W1, rout0, rout1)
    agg = _agg_call(h0, h1, s0, d0, s1, d1, z_hbm)
    return _fin_call(agg[0], agg[1], rin0, rin1,
                     b0.reshape(1, D), b1.reshape(1, D))
